# baseline (speedup 1.0000x reference)
import jax
import jax.numpy as jnp
from jax import lax
from jax.experimental import pallas as pl
from jax.experimental.pallas import tpu as pltpu
from jax.experimental.pallas import tpu_sc as plsc

NC, NS = 2, 16
NW = NC * NS
K = 4
L = 2
SPLITS = ((0, 128), (128, 72))


def _emb_body(idx_hbm, table_hbm, out_hbm, idx_v, rows_v, gsem, osem):
    wid = lax.axis_index("s") * NC + lax.axis_index("c")
    rows_per_w = idx_v.shape[0]
    row0 = wid * rows_per_w

    pltpu.sync_copy(idx_hbm.at[pl.ds(row0, rows_per_w)], idx_v)

    def start_gather(r, b):
        for off, n in SPLITS:
            pltpu.async_copy(table_hbm.at[idx_v.at[r, pl.ds(off, n)]],
                             rows_v.at[b, pl.ds(off, n)], gsem.at[b])

    def wait_gather(b):
        for off, n in SPLITS:
            pltpu.make_async_copy(table_hbm.at[idx_v.at[0, pl.ds(off, n)]],
                                  rows_v.at[b, pl.ds(off, n)],
                                  gsem.at[b]).wait()

    def start_out(r, b):
        pltpu.async_copy(rows_v.at[b], out_hbm.at[row0 + r], osem.at[b])

    def wait_out(b):
        pltpu.make_async_copy(rows_v.at[b], out_hbm.at[0], osem.at[b]).wait()

    for r in range(L):
        start_gather(r, r)
    for r in range(K - L):
        start_gather(r + L, r + L)
        wait_gather(r)
        start_out(r, r)

    def steady(r, carry):
        bg = (r + L) % K
        wait_out(bg)
        start_gather(r + L, bg)
        b = r % K
        wait_gather(b)
        start_out(r, b)
        return carry

    lax.fori_loop(K - L, rows_per_w - L, steady, 0)

    for r in range(rows_per_w - L, rows_per_w):
        wait_gather(r % K)
        start_out(r, r % K)
    for r in range(rows_per_w - K, rows_per_w):
        wait_out(r % K)


def kernel(indices, table):
    B, H = indices.shape
    D = table.shape[1]
    rows_per_w = B // NW
    idx = indices.astype(jnp.int32)

    run = pl.kernel(
        _emb_body,
        out_type=jax.ShapeDtypeStruct((B, H, D), jnp.float32),
        mesh=plsc.VectorSubcoreMesh(core_axis_name="c", subcore_axis_name="s"),
        compiler_params=pltpu.CompilerParams(use_tc_tiling_on_sc=False),
        scratch_types=[
            pltpu.VMEM((rows_per_w, H), jnp.int32),
            pltpu.VMEM((K, H, D), jnp.float32),
            pltpu.SemaphoreType.DMA((K,)),
            pltpu.SemaphoreType.DMA((K,)),
        ],
    )
    return run(idx, table)
